# baseline (speedup 1.0000x reference)
import jax
import jax.numpy as jnp
from jax.experimental import pallas as pl
from jax.experimental.pallas import tpu as pltpu

_E = 8
_T = 4096
_H = 768


def _moe_kernel(x_ref, rw_ref, rb_ref, ew_ref, eb_ref, out_ref,
                xb_ref, wmat_ref):
    e_step = pl.program_id(0)

    @pl.when(e_step == 0)
    def _router():
        x = x_ref[...]
        scores = (
            jnp.dot(x, rw_ref[...], preferred_element_type=jnp.float32)
            + rb_ref[...]
        )
        m = jnp.max(scores, axis=-1, keepdims=True)
        ex = jnp.exp(scores - m)
        probs = ex / jnp.sum(ex, axis=-1, keepdims=True)

        i0 = jnp.argmax(probs, axis=-1).reshape(-1, 1)
        p0 = jnp.max(probs, axis=-1, keepdims=True)
        iota = jax.lax.broadcasted_iota(jnp.int32, probs.shape, 1)
        masked = jnp.where(iota == i0, probs - 2.0, probs)
        i1 = jnp.argmax(masked, axis=-1).reshape(-1, 1)
        p1 = jnp.max(masked, axis=-1, keepdims=True)
        denom = p0 + p1
        wmat_ref[...] = jnp.where(iota == i0, p0 / denom, 0.0) + jnp.where(
            iota == i1, p1 / denom, 0.0
        )
        xb_ref[...] = x.astype(jnp.bfloat16)

    wb = ew_ref[0].astype(jnp.bfloat16)
    y = jnp.dot(xb_ref[...], wb, preferred_element_type=jnp.float32)
    wm = wmat_ref[...]
    sel = jax.lax.broadcasted_iota(jnp.int32, wm.shape, 1) == e_step
    w = jnp.sum(jnp.where(sel, wm, 0.0), axis=1, keepdims=True)
    contrib = w * y

    @pl.when(e_step == 0)
    def _first():
        out_ref[...] = contrib

    @pl.when(e_step > 0)
    def _rest():
        out_ref[...] = out_ref[...] + contrib

    @pl.when(e_step == _E - 1)
    def _bias():
        out_ref[...] = out_ref[...] + jnp.dot(
            wmat_ref[...], eb_ref[...], preferred_element_type=jnp.float32
        )


def kernel(x, router_w, router_b, expert_w, expert_b):
    b, s, h = x.shape
    flat_x = x.reshape(_T, _H)
    rb2 = router_b.reshape(1, -1)

    out = pl.pallas_call(
        _moe_kernel,
        grid=(_E,),
        in_specs=[
            pl.BlockSpec((_T, _H), lambda e: (0, 0)),
            pl.BlockSpec((_H, _E), lambda e: (0, 0)),
            pl.BlockSpec((1, _E), lambda e: (0, 0)),
            pl.BlockSpec((1, _H, _H), lambda e: (e, 0, 0)),
            pl.BlockSpec((_E, _H), lambda e: (0, 0)),
        ],
        out_specs=pl.BlockSpec((_T, _H), lambda e: (0, 0)),
        out_shape=jax.ShapeDtypeStruct((_T, _H), jnp.float32),
        scratch_shapes=[
            pltpu.VMEM((_T, _H), jnp.bfloat16),
            pltpu.VMEM((_T, _E), jnp.float32),
        ],
    )(flat_x, router_w, rb2, expert_w, expert_b)
    return out.reshape(b, s, h)
